# baseline (speedup 1.0000x reference)
import jax
import jax.numpy as jnp
from jax.experimental import pallas as pl
from jax.experimental.pallas import tpu as pltpu

B, T, D_NUM = 16, 2048, 256
MASK_D = 32
LAB_DIM = 16
STRAIN_DIM = 8
D_OUT = D_NUM + 4 * MASK_D + LAB_DIM + 2 * STRAIN_DIM


def _body(lab_sref, astr_sref, tstr_sref,
          num_ref, ax_ref, ay_ref, tx_ref, ty_ref,
          mean_ref, std_ref, lab_tab_ref, strain_tab_ref,
          out_ref):
    b = pl.program_id(0)
    normed = (num_ref[0] - mean_ref[0]) / std_ref[0]
    lab_vec = lab_tab_ref[pl.ds(lab_sref[b], 1), :]
    s1_vec = strain_tab_ref[pl.ds(astr_sref[b], 1), :]
    s2_vec = strain_tab_ref[pl.ds(tstr_sref[b], 1), :]
    out_ref[0] = jnp.concatenate(
        [
            normed,
            ax_ref[0], ay_ref[0], tx_ref[0], ty_ref[0],
            jnp.broadcast_to(lab_vec, (T, LAB_DIM)),
            jnp.broadcast_to(s1_vec, (T, STRAIN_DIM)),
            jnp.broadcast_to(s2_vec, (T, STRAIN_DIM)),
        ],
        axis=-1,
    )


def kernel(numeric_feats, agent_x_mask, agent_y_mask, target_x_mask,
           target_y_mask, lab_idx, agent_strain_idx, target_strain_idx,
           mean, std, emb_lab, emb_strain):
    lab_idx = lab_idx.astype(jnp.int32)
    agent_strain_idx = agent_strain_idx.astype(jnp.int32)
    target_strain_idx = target_strain_idx.astype(jnp.int32)
    mean2 = mean.reshape(1, D_NUM)
    std2 = std.reshape(1, D_NUM)

    grid_spec = pltpu.PrefetchScalarGridSpec(
        num_scalar_prefetch=3,
        grid=(B,),
        in_specs=[
            pl.BlockSpec((1, T, D_NUM), lambda b, *_: (b, 0, 0)),
            pl.BlockSpec((1, T, MASK_D), lambda b, *_: (b, 0, 0)),
            pl.BlockSpec((1, T, MASK_D), lambda b, *_: (b, 0, 0)),
            pl.BlockSpec((1, T, MASK_D), lambda b, *_: (b, 0, 0)),
            pl.BlockSpec((1, T, MASK_D), lambda b, *_: (b, 0, 0)),
            pl.BlockSpec((1, D_NUM), lambda b, *_: (0, 0)),
            pl.BlockSpec((1, D_NUM), lambda b, *_: (0, 0)),
            pl.BlockSpec(emb_lab.shape, lambda b, *_: (0, 0)),
            pl.BlockSpec(emb_strain.shape, lambda b, *_: (0, 0)),
        ],
        out_specs=pl.BlockSpec((1, T, D_OUT), lambda b, *_: (b, 0, 0)),
    )

    return pl.pallas_call(
        _body,
        grid_spec=grid_spec,
        out_shape=jax.ShapeDtypeStruct((B, T, D_OUT), jnp.float32),
    )(lab_idx, agent_strain_idx, target_strain_idx,
      numeric_feats, agent_x_mask, agent_y_mask, target_x_mask,
      target_y_mask, mean2, std2, emb_lab, emb_strain)
